# baseline (speedup 1.0000x reference)
import jax
import jax.numpy as jnp
from jax import lax
from jax.experimental import pallas as pl
from jax.experimental.pallas import tpu as pltpu
from jax.experimental.pallas import tpu_sc as plsc

N = 10000
NP = 10240
E = 320000
D = 128
NC, NS, L = 2, 16, 16
NW = NC * NS
EPW = E // NW
DE = 80
DN = EPW // DE
AE = 80
AN = EPW // AE
RPT = NP // NS
TCB = 2048


def _mesh():
    return plsc.VectorSubcoreMesh(
        core_axis_name="c", subcore_axis_name="s",
        num_cores=NC, num_subcores=NS)


def _zero_stripe(zbuf, zrows, acc_sh, s):
    def zfill(r, carry):
        for k in range(D // L):
            zbuf[r, pl.ds(k * L, L)] = jnp.zeros((L,), jnp.float32)
        return carry

    lax.fori_loop(0, zrows, zfill, 0)
    for j in range(RPT // zrows):
        pltpu.sync_copy(zbuf, acc_sh.at[pl.ds(s * RPT + j * zrows, zrows)])


def _deg_body(dst_hbm, out_hbm, dst_v, ones_v, zbuf, deg_sh, sem):
    c = lax.axis_index("c")
    s = lax.axis_index("s")
    wid = c * NS + s

    def fill(r, carry):
        for k in range(D // L):
            ones_v[r, pl.ds(k * L, L)] = jnp.ones((L,), jnp.float32)
        return carry

    lax.fori_loop(0, DE, fill, 0)
    pltpu.sync_copy(dst_hbm.at[wid], dst_v)
    _zero_stripe(zbuf, DE, deg_sh, s)
    plsc.subcore_barrier()

    K = 8
    for i in range(K):
        pltpu.async_copy(ones_v, deg_sh.at[dst_v.at[i]], sem, add=True)

    def body(i, carry):
        pltpu.make_async_copy(ones_v, deg_sh.at[dst_v.at[0]], sem).wait()
        pltpu.async_copy(ones_v, deg_sh.at[dst_v.at[i + K]], sem, add=True)
        return carry

    lax.fori_loop(0, DN - K, body, 0)
    for i in range(K):
        pltpu.make_async_copy(ones_v, deg_sh.at[dst_v.at[0]], sem).wait()
    plsc.subcore_barrier()
    pltpu.sync_copy(deg_sh.at[pl.ds(s * RPT, RPT)],
                    out_hbm.at[pl.ds(c * NP + s * RPT, RPT)])


def _deg_call(dst3):
    f = pl.kernel(
        _deg_body,
        out_type=jax.ShapeDtypeStruct((NC * NP, D), jnp.float32),
        mesh=_mesh(),
        scratch_types=[
            pltpu.VMEM((DN, DE), jnp.int32),
            pltpu.VMEM((DE, D), jnp.float32),
            pltpu.VMEM((DE, D), jnp.float32),
            pltpu.VMEM_SHARED((NP, D), jnp.float32),
            pltpu.SemaphoreType.DMA,
        ],
    )
    return f(dst3)


def _agg_body(hs_hbm, src_hbm, dst_hbm, out_hbm,
              dst_v, src0, src1, rows0, rows1, agg_sh,
              sem0, sem1, semi0, semi1, ssem0, ssem1):
    c = lax.axis_index("c")
    s = lax.axis_index("s")
    wid = c * NS + s

    pltpu.sync_copy(dst_hbm.at[wid], dst_v)
    _zero_stripe(rows0, AE, agg_sh, s)
    plsc.subcore_barrier()

    ebase = wid * EPW
    pltpu.sync_copy(src_hbm.at[pl.ds(ebase, AE)], src0)
    pltpu.async_copy(hs_hbm.at[src0], rows0, sem0)
    pltpu.sync_copy(src_hbm.at[pl.ds(ebase + AE, AE)], src1)

    def wait(buf, sem):
        pltpu.make_async_copy(hs_hbm.at[src0], buf, sem).wait()

    def wait_idx(buf, sem):
        pltpu.make_async_copy(src_hbm.at[pl.ds(ebase, AE)], buf, sem).wait()

    def wait_scat(buf, sem):
        pltpu.make_async_copy(buf, agg_sh.at[dst_v.at[0]], sem).wait()

    def body(t, carry):
        g = 2 * t

        @pl.when(t > 0)
        def _():
            wait_scat(rows1, ssem1)

        pltpu.async_copy(hs_hbm.at[src1], rows1, sem1)
        wait(rows0, sem0)
        pltpu.async_copy(
            src_hbm.at[pl.ds(ebase + (g + 2) * AE, AE)], src0, semi0)
        pltpu.async_copy(rows0, agg_sh.at[dst_v.at[g]],
                         ssem0, add=True)
        wait_idx(src0, semi0)
        wait(rows1, sem1)

        @pl.when(g + 3 < AN)
        def _():
            pltpu.async_copy(
                src_hbm.at[pl.ds(ebase + (g + 3) * AE, AE)], src1, semi1)

        pltpu.async_copy(rows1, agg_sh.at[dst_v.at[g + 1]],
                         ssem1, add=True)
        wait_scat(rows0, ssem0)
        pltpu.async_copy(hs_hbm.at[src0], rows0, sem0)

        @pl.when(g + 3 < AN)
        def _():
            wait_idx(src1, semi1)

        return carry

    lax.fori_loop(0, (AN - 1) // 2, body, 0)
    wait_scat(rows1, ssem1)
    wait(rows0, sem0)
    pltpu.sync_copy(rows0, agg_sh.at[dst_v.at[AN - 1]], add=True)

    plsc.subcore_barrier()
    pltpu.sync_copy(agg_sh.at[pl.ds(s * RPT, RPT)],
                    out_hbm.at[pl.ds(c * NP + s * RPT, RPT)])


def _agg_call(hs, src, dst3):
    f = pl.kernel(
        _agg_body,
        out_type=jax.ShapeDtypeStruct((NC * NP, D), jnp.float32),
        mesh=_mesh(),
        scratch_types=[
            pltpu.VMEM((AN, AE), jnp.int32),
            pltpu.VMEM((AE,), jnp.int32),
            pltpu.VMEM((AE,), jnp.int32),
            pltpu.VMEM((AE, D), jnp.float32),
            pltpu.VMEM((AE, D), jnp.float32),
            pltpu.VMEM_SHARED((NP, D), jnp.float32),
            pltpu.SemaphoreType.DMA,
            pltpu.SemaphoreType.DMA,
            pltpu.SemaphoreType.DMA,
            pltpu.SemaphoreType.DMA,
            pltpu.SemaphoreType.DMA,
            pltpu.SemaphoreType.DMA,
        ],
    )
    return f(hs, src, dst3)


def _tc_a_body(x_ref, p_ref, w_ref, hs_ref, dis_ref):
    deg = 1.0 + p_ref[0, :, 0:1] + p_ref[1, :, 0:1]
    dis = lax.rsqrt(deg)
    h = jnp.dot(x_ref[...], w_ref[...], preferred_element_type=jnp.float32)
    hs_ref[...] = dis * h
    dis_ref[...] = dis


def _tc_a(x_pad, degp, W1):
    return pl.pallas_call(
        _tc_a_body,
        grid=(NP // TCB,),
        in_specs=[
            pl.BlockSpec((TCB, D), lambda i: (i, 0)),
            pl.BlockSpec((2, TCB, D), lambda i: (0, i, 0)),
            pl.BlockSpec((D, D), lambda i: (0, 0)),
        ],
        out_specs=[
            pl.BlockSpec((TCB, D), lambda i: (i, 0)),
            pl.BlockSpec((TCB, 1), lambda i: (i, 0)),
        ],
        out_shape=[
            jax.ShapeDtypeStruct((NP, D), jnp.float32),
            jax.ShapeDtypeStruct((NP, 1), jnp.float32),
        ],
    )(x_pad, degp, W1)


def _tc_b_body(p_ref, hs_ref, dis_ref, w_ref, b_ref, out_ref):
    agg = p_ref[0] + p_ref[1] + hs_ref[...]
    o1 = jnp.maximum(dis_ref[...] * agg + b_ref[...], 0.0)
    out_ref[...] = dis_ref[...] * jnp.dot(
        o1, w_ref[...], preferred_element_type=jnp.float32)


def _tc_b(p1, hs1, dis, W2, b1):
    return pl.pallas_call(
        _tc_b_body,
        grid=(NP // TCB,),
        in_specs=[
            pl.BlockSpec((2, TCB, D), lambda i: (0, i, 0)),
            pl.BlockSpec((TCB, D), lambda i: (i, 0)),
            pl.BlockSpec((TCB, 1), lambda i: (i, 0)),
            pl.BlockSpec((D, D), lambda i: (0, 0)),
            pl.BlockSpec((1, D), lambda i: (0, 0)),
        ],
        out_specs=pl.BlockSpec((TCB, D), lambda i: (i, 0)),
        out_shape=jax.ShapeDtypeStruct((NP, D), jnp.float32),
    )(p1, hs1, dis, W2, b1)


def _tc_c_body(p_ref, hs_ref, dis_ref, b_ref, out_ref):
    agg = p_ref[0] + p_ref[1] + hs_ref[...]
    out_ref[...] = dis_ref[...] * agg + b_ref[...]


def _tc_c(p2, hs2, dis, b2):
    return pl.pallas_call(
        _tc_c_body,
        grid=(NP // TCB,),
        in_specs=[
            pl.BlockSpec((2, TCB, D), lambda i: (0, i, 0)),
            pl.BlockSpec((TCB, D), lambda i: (i, 0)),
            pl.BlockSpec((TCB, 1), lambda i: (i, 0)),
            pl.BlockSpec((1, D), lambda i: (0, 0)),
        ],
        out_specs=pl.BlockSpec((TCB, D), lambda i: (i, 0)),
        out_shape=jax.ShapeDtypeStruct((NP, D), jnp.float32),
    )(p2, hs2, dis, b2)


def kernel(x, edge_index, W1, b1, W2, b2):
    ei = edge_index.astype(jnp.int32)
    src = ei[0]
    dst3d = ei[1].reshape(NW, DN, DE)
    dst3a = ei[1].reshape(NW, AN, AE)
    x_pad = jnp.zeros((NP, D), jnp.float32).at[:N].set(x)

    degp = _deg_call(dst3d).reshape(NC, NP, D)
    hs1, dis = _tc_a(x_pad, degp, W1)
    p1 = _agg_call(hs1, src, dst3a).reshape(NC, NP, D)
    hs2 = _tc_b(p1, hs1, dis, W2, b1.reshape(1, D))
    p2 = _agg_call(hs2, src, dst3a).reshape(NC, NP, D)
    out = _tc_c(p2, hs2, dis, b2.reshape(1, D))
    return out[:N]


# speedup vs baseline: 1.0113x; 1.0113x over previous
import jax
import jax.numpy as jnp
from jax import lax
from jax.experimental import pallas as pl
from jax.experimental.pallas import tpu as pltpu
from jax.experimental.pallas import tpu_sc as plsc

N = 10000
NP = 10240
E = 320000
D = 128
NC, NS, L = 2, 16, 16
NW = NC * NS
EPW = E // NW
DE = 80
DN = EPW // DE
AE = 80
AN = EPW // AE
RPT = NP // NS
TCB = 2048


def _mesh():
    return plsc.VectorSubcoreMesh(
        core_axis_name="c", subcore_axis_name="s",
        num_cores=NC, num_subcores=NS)


def _zero_stripe(zbuf, zrows, acc_sh, s):
    def zfill(r, carry):
        for k in range(D // L):
            zbuf[r, pl.ds(k * L, L)] = jnp.zeros((L,), jnp.float32)
        return carry

    lax.fori_loop(0, zrows, zfill, 0)
    for j in range(RPT // zrows):
        pltpu.sync_copy(zbuf, acc_sh.at[pl.ds(s * RPT + j * zrows, zrows)])


def _deg_body(dst_hbm, out_hbm, dst_v, ones_v, zbuf, deg_sh, sem):
    c = lax.axis_index("c")
    s = lax.axis_index("s")
    wid = c * NS + s

    def fill(r, carry):
        for k in range(D // L):
            ones_v[r, pl.ds(k * L, L)] = jnp.ones((L,), jnp.float32)
        return carry

    lax.fori_loop(0, DE, fill, 0)
    pltpu.sync_copy(dst_hbm.at[wid], dst_v)
    _zero_stripe(zbuf, DE, deg_sh, s)
    plsc.subcore_barrier()

    K = 8
    for i in range(K):
        pltpu.async_copy(ones_v, deg_sh.at[dst_v.at[i]], sem, add=True)

    def body(i, carry):
        pltpu.make_async_copy(ones_v, deg_sh.at[dst_v.at[0]], sem).wait()
        pltpu.async_copy(ones_v, deg_sh.at[dst_v.at[i + K]], sem, add=True)
        return carry

    lax.fori_loop(0, DN - K, body, 0)
    for i in range(K):
        pltpu.make_async_copy(ones_v, deg_sh.at[dst_v.at[0]], sem).wait()
    plsc.subcore_barrier()
    pltpu.sync_copy(deg_sh.at[pl.ds(s * RPT, RPT)],
                    out_hbm.at[pl.ds(c * NP + s * RPT, RPT)])


def _deg_call(dst3):
    f = pl.kernel(
        _deg_body,
        out_type=jax.ShapeDtypeStruct((NC * NP, D), jnp.float32),
        mesh=_mesh(),
        scratch_types=[
            pltpu.VMEM((DN, DE), jnp.int32),
            pltpu.VMEM((DE, D), jnp.float32),
            pltpu.VMEM((DE, D), jnp.float32),
            pltpu.VMEM_SHARED((NP, D), jnp.float32),
            pltpu.SemaphoreType.DMA,
        ],
    )
    return f(dst3)


def _agg_body(hs_hbm, src_hbm, dst_hbm, out_hbm,
              dst_v, src0, src1, rows0, rows1, agg_sh,
              sem0, sem1, semi0, semi1):
    c = lax.axis_index("c")
    s = lax.axis_index("s")
    wid = c * NS + s

    pltpu.sync_copy(dst_hbm.at[wid], dst_v)
    _zero_stripe(rows0, AE, agg_sh, s)
    plsc.subcore_barrier()

    ebase = wid * EPW
    pltpu.sync_copy(src_hbm.at[pl.ds(ebase, AE)], src0)
    pltpu.async_copy(hs_hbm.at[src0], rows0, sem0)
    pltpu.sync_copy(src_hbm.at[pl.ds(ebase + AE, AE)], src1)

    def wait(buf, sem):
        pltpu.make_async_copy(hs_hbm.at[src0], buf, sem).wait()

    def wait_idx(buf, sem):
        pltpu.make_async_copy(src_hbm.at[pl.ds(ebase, AE)], buf, sem).wait()

    def body(t, carry):
        g = 2 * t
        pltpu.async_copy(hs_hbm.at[src1], rows1, sem1)
        wait(rows0, sem0)
        pltpu.async_copy(
            src_hbm.at[pl.ds(ebase + (g + 2) * AE, AE)], src0, semi0)
        pltpu.sync_copy(rows0, agg_sh.at[dst_v.at[g]], add=True)
        wait_idx(src0, semi0)
        pltpu.async_copy(hs_hbm.at[src0], rows0, sem0)
        wait(rows1, sem1)

        @pl.when(g + 3 < AN)
        def _():
            pltpu.async_copy(
                src_hbm.at[pl.ds(ebase + (g + 3) * AE, AE)], src1, semi1)

        pltpu.sync_copy(rows1, agg_sh.at[dst_v.at[g + 1]], add=True)

        @pl.when(g + 3 < AN)
        def _():
            wait_idx(src1, semi1)

        return carry

    lax.fori_loop(0, (AN - 1) // 2, body, 0)
    wait(rows0, sem0)
    pltpu.sync_copy(rows0, agg_sh.at[dst_v.at[AN - 1]], add=True)

    plsc.subcore_barrier()
    pltpu.sync_copy(agg_sh.at[pl.ds(s * RPT, RPT)],
                    out_hbm.at[pl.ds(c * NP + s * RPT, RPT)])


def _agg_call(hs, src, dst3):
    f = pl.kernel(
        _agg_body,
        out_type=jax.ShapeDtypeStruct((NC * NP, D), jnp.float32),
        mesh=_mesh(),
        scratch_types=[
            pltpu.VMEM((AN, AE), jnp.int32),
            pltpu.VMEM((AE,), jnp.int32),
            pltpu.VMEM((AE,), jnp.int32),
            pltpu.VMEM((AE, D), jnp.float32),
            pltpu.VMEM((AE, D), jnp.float32),
            pltpu.VMEM_SHARED((NP, D), jnp.float32),
            pltpu.SemaphoreType.DMA,
            pltpu.SemaphoreType.DMA,
            pltpu.SemaphoreType.DMA,
            pltpu.SemaphoreType.DMA,
        ],
    )
    return f(hs, src, dst3)


def _tc_a_body(x_ref, p_ref, w_ref, hs_ref, dis_ref):
    deg = 1.0 + p_ref[0, :, 0:1] + p_ref[1, :, 0:1]
    dis = lax.rsqrt(deg)
    h = jnp.dot(x_ref[...], w_ref[...], preferred_element_type=jnp.float32)
    hs_ref[...] = dis * h
    dis_ref[...] = dis


def _tc_a(x_pad, degp, W1):
    return pl.pallas_call(
        _tc_a_body,
        grid=(NP // TCB,),
        in_specs=[
            pl.BlockSpec((TCB, D), lambda i: (i, 0)),
            pl.BlockSpec((2, TCB, D), lambda i: (0, i, 0)),
            pl.BlockSpec((D, D), lambda i: (0, 0)),
        ],
        out_specs=[
            pl.BlockSpec((TCB, D), lambda i: (i, 0)),
            pl.BlockSpec((TCB, 1), lambda i: (i, 0)),
        ],
        out_shape=[
            jax.ShapeDtypeStruct((NP, D), jnp.float32),
            jax.ShapeDtypeStruct((NP, 1), jnp.float32),
        ],
    )(x_pad, degp, W1)


def _tc_b_body(p_ref, hs_ref, dis_ref, w_ref, b_ref, out_ref):
    agg = p_ref[0] + p_ref[1] + hs_ref[...]
    o1 = jnp.maximum(dis_ref[...] * agg + b_ref[...], 0.0)
    out_ref[...] = dis_ref[...] * jnp.dot(
        o1, w_ref[...], preferred_element_type=jnp.float32)


def _tc_b(p1, hs1, dis, W2, b1):
    return pl.pallas_call(
        _tc_b_body,
        grid=(NP // TCB,),
        in_specs=[
            pl.BlockSpec((2, TCB, D), lambda i: (0, i, 0)),
            pl.BlockSpec((TCB, D), lambda i: (i, 0)),
            pl.BlockSpec((TCB, 1), lambda i: (i, 0)),
            pl.BlockSpec((D, D), lambda i: (0, 0)),
            pl.BlockSpec((1, D), lambda i: (0, 0)),
        ],
        out_specs=pl.BlockSpec((TCB, D), lambda i: (i, 0)),
        out_shape=jax.ShapeDtypeStruct((NP, D), jnp.float32),
    )(p1, hs1, dis, W2, b1)


def _tc_c_body(p_ref, hs_ref, dis_ref, b_ref, out_ref):
    agg = p_ref[0] + p_ref[1] + hs_ref[...]
    out_ref[...] = dis_ref[...] * agg + b_ref[...]


def _tc_c(p2, hs2, dis, b2):
    return pl.pallas_call(
        _tc_c_body,
        grid=(NP // TCB,),
        in_specs=[
            pl.BlockSpec((2, TCB, D), lambda i: (0, i, 0)),
            pl.BlockSpec((TCB, D), lambda i: (i, 0)),
            pl.BlockSpec((TCB, 1), lambda i: (i, 0)),
            pl.BlockSpec((1, D), lambda i: (0, 0)),
        ],
        out_specs=pl.BlockSpec((TCB, D), lambda i: (i, 0)),
        out_shape=jax.ShapeDtypeStruct((NP, D), jnp.float32),
    )(p2, hs2, dis, b2)


def kernel(x, edge_index, W1, b1, W2, b2):
    ei = edge_index.astype(jnp.int32)
    src = ei[0]
    dst3d = ei[1].reshape(NW, DN, DE)
    dst3a = ei[1].reshape(NW, AN, AE)
    x_pad = jnp.zeros((NP, D), jnp.float32).at[:N].set(x)

    degp = _deg_call(dst3d).reshape(NC, NP, D)
    hs1, dis = _tc_a(x_pad, degp, W1)
    p1 = _agg_call(hs1, src, dst3a).reshape(NC, NP, D)
    hs2 = _tc_b(p1, hs1, dis, W2, b1.reshape(1, D))
    p2 = _agg_call(hs2, src, dst3a).reshape(NC, NP, D)
    out = _tc_c(p2, hs2, dis, b2.reshape(1, D))
    return out[:N]
